# Initial kernel scaffold; baseline (speedup 1.0000x reference)
import jax
import jax.numpy as jnp
from jax.experimental import pallas as pl


def kernel(edge_lists, node_states, W, b):
    raise NotImplementedError("write your pallas kernel here")


# speedup vs baseline: 2.9239x; 2.9239x over previous
import functools

import jax
import jax.numpy as jnp
from jax import lax
from jax.experimental import pallas as pl
from jax.experimental.pallas import tpu as pltpu
from jax.experimental.pallas import tpu_sc as plsc

T = 4
DIM = 128
N = 10000
M = 80000

NC = 2
NS = 16
NW = NC * NS
CH = 128
GRP = 16
E = T * M
NCH = 80
EPW = NCH * CH
ZROW = T * N

NB = 10
MM_BLK = N // NB
N_PAD = 10240
RPT = N_PAD // NS


def _mm_body(ns_ref, w_ref, b_ref, out_ref):
    i = pl.program_id(0)

    @pl.when(i < T * NB)
    def _():
        out_ref[...] = lax.dot_general(
            ns_ref[...], w_ref[...], (((1,), (1,)), ((), ())),
            preferred_element_type=jnp.float32) + b_ref[0]

    @pl.when(i >= T * NB)
    def _():
        out_ref[...] = jnp.zeros_like(out_ref)


def _build_table(node_states, W, b2d):
    return pl.pallas_call(
        _mm_body,
        grid=(T * NB + 1,),
        in_specs=[
            pl.BlockSpec((MM_BLK, DIM), lambda i: (i % NB, 0)),
            pl.BlockSpec((DIM, DIM), lambda i: (jnp.minimum(i // NB, T - 1), 0)),
            pl.BlockSpec((1, 1, DIM), lambda i: (jnp.minimum(i // NB, T - 1), 0, 0)),
        ],
        out_specs=pl.BlockSpec((MM_BLK, DIM), lambda i: (i, 0)),
        out_shape=jax.ShapeDtypeStruct(((T * NB + 1) * MM_BLK, DIM), jnp.float32),
    )(node_states, W, b2d)


def _sc_body(table_hbm, src_hbm, tgt_hbm, out_hbm,
             src_v, tgt_v, buf0, buf1, acc, sem0, sem1):
    cid = lax.axis_index("c")
    sid = lax.axis_index("s")
    wid = cid * NS + sid

    pltpu.sync_copy(src_hbm.at[wid], src_v)

    row0 = sid * RPT
    pltpu.sync_copy(table_hbm.at[pl.ds(ZROW, RPT)], acc.at[pl.ds(row0, RPT)])
    plsc.subcore_barrier()

    bufs = (buf0, buf1)
    sems = (sem0, sem1)

    pltpu.async_copy(table_hbm.at[src_v.at[0]], buf0, sem0)

    def group(g, carry):
        pltpu.sync_copy(tgt_hbm.at[wid, pl.ds(g * GRP, GRP)], tgt_v)
        for p in range(GRP):
            j = g * GRP + p
            pltpu.make_async_copy(
                table_hbm.at[pl.ds(0, CH)], bufs[p % 2], sems[p % 2]).wait()

            @pl.when(j + 1 < NCH)
            def _():
                pltpu.async_copy(
                    table_hbm.at[src_v.at[j + 1]],
                    bufs[(p + 1) % 2], sems[(p + 1) % 2])

            pltpu.sync_copy(bufs[p % 2], acc.at[tgt_v.at[p]], add=True)
        return carry

    lax.fori_loop(0, NCH // GRP, group, 0)

    plsc.subcore_barrier()
    pltpu.sync_copy(acc.at[pl.ds(row0, RPT)],
                    out_hbm.at[cid, pl.ds(row0, RPT)])


_sc_scatter = functools.partial(
    pl.kernel,
    out_type=jax.ShapeDtypeStruct((NC, N_PAD, DIM), jnp.float32),
    mesh=plsc.VectorSubcoreMesh(core_axis_name="c", subcore_axis_name="s"),
    scratch_types=[
        pltpu.VMEM((NCH, CH), jnp.int32),
        pltpu.VMEM((GRP, CH), jnp.int32),
        pltpu.VMEM((CH, DIM), jnp.float32),
        pltpu.VMEM((CH, DIM), jnp.float32),
        pltpu.VMEM_SHARED((N_PAD, DIM), jnp.float32),
        pltpu.SemaphoreType.DMA,
        pltpu.SemaphoreType.DMA,
    ],
)(_sc_body)


def _merge_body(p_ref, o_ref):
    o_ref[...] = p_ref[0] + p_ref[1]


def _merge(parts):
    return pl.pallas_call(
        _merge_body,
        grid=(NB,),
        in_specs=[pl.BlockSpec((NC, MM_BLK, DIM), lambda i: (0, i, 0))],
        out_specs=pl.BlockSpec((MM_BLK, DIM), lambda i: (i, 0)),
        out_shape=jax.ShapeDtypeStruct((N, DIM), jnp.float32),
    )(parts)


def kernel(edge_lists, node_states, W, b):
    edge_lists = edge_lists.astype(jnp.int32)
    table = _build_table(node_states, W, b.reshape(T, 1, DIM))

    src = edge_lists[:, :, 0] + (jnp.arange(T, dtype=jnp.int32) * N)[:, None]
    tgt = edge_lists[:, :, 1]
    pad = NW * EPW - E
    src_w = jnp.concatenate(
        [src.reshape(-1), jnp.full((pad,), ZROW, jnp.int32)]).reshape(NW, NCH, CH)
    tgt_w = jnp.concatenate(
        [tgt.reshape(-1), jnp.zeros((pad,), jnp.int32)]).reshape(NW, NCH, CH)

    parts = _sc_scatter(table, src_w, tgt_w)
    return _merge(parts)


# speedup vs baseline: 7.6521x; 2.6171x over previous
import functools

import jax
import jax.numpy as jnp
from jax import lax
from jax.experimental import pallas as pl
from jax.experimental.pallas import tpu as pltpu
from jax.experimental.pallas import tpu_sc as plsc

T = 4
DIM = 128
N = 10000
M = 80000

NC = 2
NS = 16
NW = NC * NS
CH = 128
GRP = 16
E = T * M
NCH = 80
EPW = NCH * CH
ZROW = T * N

NB = 10
MM_BLK = N // NB
N_PAD = 10240
RPT = N_PAD // NS


def _mm_body(ns_ref, w_ref, b_ref, out_ref):
    i = pl.program_id(0)

    @pl.when(i < T * NB)
    def _():
        out_ref[...] = lax.dot_general(
            ns_ref[...], w_ref[...], (((1,), (1,)), ((), ())),
            preferred_element_type=jnp.float32) + b_ref[0]

    @pl.when(i >= T * NB)
    def _():
        out_ref[...] = jnp.zeros_like(out_ref)


def _build_table(node_states, W, b2d):
    return pl.pallas_call(
        _mm_body,
        grid=(T * NB + 1,),
        in_specs=[
            pl.BlockSpec((MM_BLK, DIM), lambda i: (i % NB, 0)),
            pl.BlockSpec((DIM, DIM), lambda i: (jnp.minimum(i // NB, T - 1), 0)),
            pl.BlockSpec((1, 1, DIM), lambda i: (jnp.minimum(i // NB, T - 1), 0, 0)),
        ],
        out_specs=pl.BlockSpec((MM_BLK, DIM), lambda i: (i, 0)),
        out_shape=jax.ShapeDtypeStruct(((T * NB + 1) * MM_BLK, DIM), jnp.float32),
    )(node_states, W, b2d)


def _sc_body(table_hbm, src_hbm, tgt_hbm, out_hbm,
             src_v, tgt_v, buf0, buf1, acc, sem0, sem1):
    cid = lax.axis_index("c")
    sid = lax.axis_index("s")
    wid = cid * NS + sid

    pltpu.sync_copy(src_hbm.at[wid], src_v)

    row0 = sid * RPT
    pltpu.sync_copy(table_hbm.at[pl.ds(ZROW, RPT)], acc.at[pl.ds(row0, RPT)])
    plsc.subcore_barrier()

    bufs = (buf0, buf1)
    sems = (sem0, sem1)

    pltpu.async_copy(table_hbm.at[src_v.at[0]], buf0, sem0)

    def group(g, carry):
        pltpu.sync_copy(tgt_hbm.at[wid, pl.ds(g * GRP, GRP)], tgt_v)
        for p in range(GRP):
            j = g * GRP + p
            pltpu.make_async_copy(
                table_hbm.at[pl.ds(0, CH)], bufs[p % 2], sems[p % 2]).wait()

            @pl.when(j + 1 < NCH)
            def _():
                pltpu.async_copy(
                    table_hbm.at[src_v.at[j + 1]],
                    bufs[(p + 1) % 2], sems[(p + 1) % 2])

            pltpu.sync_copy(bufs[p % 2], acc.at[tgt_v.at[p]], add=True)
        return carry

    lax.fori_loop(0, NCH // GRP, group, 0)

    plsc.subcore_barrier()
    pltpu.sync_copy(acc.at[pl.ds(row0, RPT)],
                    out_hbm.at[cid, pl.ds(row0, RPT)])


_sc_scatter = functools.partial(
    pl.kernel,
    out_type=jax.ShapeDtypeStruct((NC, N_PAD, DIM), jnp.float32),
    mesh=plsc.VectorSubcoreMesh(core_axis_name="c", subcore_axis_name="s"),
    scratch_types=[
        pltpu.VMEM((NCH, CH), jnp.int32),
        pltpu.VMEM((GRP, CH), jnp.int32),
        pltpu.VMEM((CH, DIM), jnp.float32),
        pltpu.VMEM((CH, DIM), jnp.float32),
        pltpu.VMEM_SHARED((N_PAD, DIM), jnp.float32),
        pltpu.SemaphoreType.DMA,
        pltpu.SemaphoreType.DMA,
    ],
)(_sc_body)


def _merge_body(p_ref, o_ref):
    o_ref[...] = p_ref[0] + p_ref[1]


def _merge(parts):
    return pl.pallas_call(
        _merge_body,
        grid=(NB,),
        in_specs=[pl.BlockSpec((NC, MM_BLK, DIM), lambda i: (0, i, 0))],
        out_specs=pl.BlockSpec((MM_BLK, DIM), lambda i: (i, 0)),
        out_shape=jax.ShapeDtypeStruct((N, DIM), jnp.float32),
    )(parts)


def kernel(edge_lists, node_states, W, b):
    edge_lists = edge_lists.astype(jnp.int32)
    table = _build_table(node_states, W, b.reshape(T, 1, DIM))

    src = edge_lists[:, :, 0] + (jnp.arange(T, dtype=jnp.int32) * N)[:, None]
    tgt = edge_lists[:, :, 1]
    pad = NW * EPW - E
    pad_ids = jnp.arange(pad, dtype=jnp.int32)
    src_w = jnp.concatenate(
        [src.reshape(-1), ZROW + pad_ids % MM_BLK]).reshape(NW, NCH, CH)
    tgt_w = jnp.concatenate(
        [tgt.reshape(-1), pad_ids % N]).reshape(NW, NCH, CH)

    parts = _sc_scatter(table, src_w, tgt_w)
    return _merge(parts)


# speedup vs baseline: 7.6910x; 1.0051x over previous
import functools

import jax
import jax.numpy as jnp
from jax import lax
from jax.experimental import pallas as pl
from jax.experimental.pallas import tpu as pltpu
from jax.experimental.pallas import tpu_sc as plsc

T = 4
DIM = 128
N = 10000
M = 80000

NC = 2
NS = 16
NW = NC * NS
CH = 128
GRP = 16
E = T * M
NCH = 80
EPW = NCH * CH

NB = 10
MM_BLK = N // NB
NROWS = (NB + 1) * MM_BLK
ZROW = N
N_PAD = 10240
RPT = N_PAD // NS


def _mm_body(ns_ref, w_ref, b_ref, out_ref):
    i = pl.program_id(0)

    @pl.when(i < NB)
    def _():
        out_ref[...] = lax.dot_general(
            ns_ref[...], w_ref[...], (((1,), (1,)), ((), ())),
            preferred_element_type=jnp.float32) + b_ref[0]

    @pl.when(i >= NB)
    def _():
        out_ref[...] = jnp.zeros_like(out_ref)


def _build_table(node_states, W, b2d):
    return pl.pallas_call(
        _mm_body,
        grid=(NB + 1, T),
        in_specs=[
            pl.BlockSpec((MM_BLK, DIM), lambda i, t: (jnp.minimum(i, NB - 1), 0)),
            pl.BlockSpec((DIM, DIM), lambda i, t: (t, 0)),
            pl.BlockSpec((1, 1, DIM), lambda i, t: (t, 0, 0)),
        ],
        out_specs=pl.BlockSpec((MM_BLK, DIM), lambda i, t: (t * (NB + 1) + i, 0)),
        out_shape=jax.ShapeDtypeStruct((T * NROWS, DIM), jnp.float32),
    )(node_states, W, b2d)


def _sc_body(table_hbm, src_hbm, tgt_hbm, out_hbm,
             src_v, tgt_v, buf0, buf1, acc, sem0, sem1, semt):
    cid = lax.axis_index("c")
    sid = lax.axis_index("s")
    wid = cid * NS + sid

    pltpu.sync_copy(src_hbm.at[wid], src_v)
    pltpu.sync_copy(tgt_hbm.at[wid, pl.ds(0, GRP)], tgt_v.at[0])

    row0 = sid * RPT
    pltpu.sync_copy(table_hbm.at[pl.ds(ZROW, RPT)], acc.at[pl.ds(row0, RPT)])
    plsc.subcore_barrier()

    bufs = (buf0, buf1)
    sems = (sem0, sem1)
    ngrp = NCH // GRP

    pltpu.async_copy(table_hbm.at[src_v.at[0]], buf0, sem0)

    def group(g, carry):
        @pl.when(g + 1 < ngrp)
        def _():
            pltpu.async_copy(
                tgt_hbm.at[wid, pl.ds((g + 1) * GRP, GRP)],
                tgt_v.at[(g + 1) % 2], semt)

        for p in range(GRP):
            j = g * GRP + p
            pltpu.make_async_copy(
                table_hbm.at[pl.ds(0, CH)], bufs[p % 2], sems[p % 2]).wait()

            @pl.when(j + 1 < NCH)
            def _():
                pltpu.async_copy(
                    table_hbm.at[src_v.at[j + 1]],
                    bufs[(p + 1) % 2], sems[(p + 1) % 2])

            pltpu.sync_copy(bufs[p % 2], acc.at[tgt_v.at[g % 2, p]], add=True)

        @pl.when(g + 1 < ngrp)
        def _():
            pltpu.make_async_copy(
                tgt_hbm.at[wid, pl.ds(0, GRP)], tgt_v.at[(g + 1) % 2],
                semt).wait()
        return carry

    lax.fori_loop(0, ngrp, group, 0)

    plsc.subcore_barrier()
    pltpu.sync_copy(acc.at[pl.ds(row0, RPT)],
                    out_hbm.at[cid, pl.ds(row0, RPT)])


_sc_scatter = functools.partial(
    pl.kernel,
    out_type=jax.ShapeDtypeStruct((NC, N_PAD, DIM), jnp.float32),
    mesh=plsc.VectorSubcoreMesh(core_axis_name="c", subcore_axis_name="s"),
    scratch_types=[
        pltpu.VMEM((NCH, CH), jnp.int32),
        pltpu.VMEM((2, GRP, CH), jnp.int32),
        pltpu.VMEM((CH, DIM), jnp.float32),
        pltpu.VMEM((CH, DIM), jnp.float32),
        pltpu.VMEM_SHARED((N_PAD, DIM), jnp.float32),
        pltpu.SemaphoreType.DMA,
        pltpu.SemaphoreType.DMA,
        pltpu.SemaphoreType.DMA,
    ],
)(_sc_body)


def _merge_body(p_ref, o_ref):
    o_ref[...] = p_ref[0] + p_ref[1]


def _merge(parts):
    return pl.pallas_call(
        _merge_body,
        grid=(NB,),
        in_specs=[pl.BlockSpec((NC, MM_BLK, DIM), lambda i: (0, i, 0))],
        out_specs=pl.BlockSpec((MM_BLK, DIM), lambda i: (i, 0)),
        out_shape=jax.ShapeDtypeStruct((N, DIM), jnp.float32),
    )(parts)


def kernel(edge_lists, node_states, W, b):
    edge_lists = edge_lists.astype(jnp.int32)
    table = _build_table(node_states, W, b.reshape(T, 1, DIM))

    src = edge_lists[:, :, 0] + (jnp.arange(T, dtype=jnp.int32) * NROWS)[:, None]
    tgt = edge_lists[:, :, 1]
    pad = NW * EPW - E
    pad_ids = jnp.arange(pad, dtype=jnp.int32)
    src_w = jnp.concatenate(
        [src.reshape(-1), ZROW + pad_ids % MM_BLK]).reshape(NW, NCH, CH)
    tgt_w = jnp.concatenate(
        [tgt.reshape(-1), pad_ids % N]).reshape(NW, NCH, CH)

    parts = _sc_scatter(table, src_w, tgt_w)
    return _merge(parts)


# speedup vs baseline: 8.6737x; 1.1278x over previous
import functools

import jax
import jax.numpy as jnp
from jax import lax
from jax.experimental import pallas as pl
from jax.experimental.pallas import tpu as pltpu
from jax.experimental.pallas import tpu_sc as plsc

T = 4
DIM = 128
N = 10000
M = 80000

NC = 2
NS = 16
NW = NC * NS
CH = 128
GRP = 16
E = T * M
NCH = 80
EPW = NCH * CH

NB = 10
MM_BLK = N // NB
NROWS = (NB + 1) * MM_BLK
ZROW = N
N_PAD = 10240
RPT = N_PAD // NS


def _mm_body(ns_ref, w_ref, b_ref, out_ref):
    i = pl.program_id(0)

    @pl.when(i < NB)
    def _():
        out_ref[...] = lax.dot_general(
            ns_ref[...], w_ref[...], (((1,), (1,)), ((), ())),
            preferred_element_type=jnp.float32) + b_ref[0]

    @pl.when(i >= NB)
    def _():
        out_ref[...] = jnp.zeros_like(out_ref)


def _build_table(node_states, W, b2d):
    return pl.pallas_call(
        _mm_body,
        grid=(NB + 1, T),
        in_specs=[
            pl.BlockSpec((MM_BLK, DIM), lambda i, t: (jnp.minimum(i, NB - 1), 0)),
            pl.BlockSpec((DIM, DIM), lambda i, t: (t, 0)),
            pl.BlockSpec((1, 1, DIM), lambda i, t: (t, 0, 0)),
        ],
        out_specs=pl.BlockSpec((MM_BLK, DIM), lambda i, t: (t * (NB + 1) + i, 0)),
        out_shape=jax.ShapeDtypeStruct((T * NROWS, DIM), jnp.float32),
    )(node_states, W, b2d)


def _sc_body(table_hbm, src_hbm, tgt_hbm, out_hbm,
             src_v, tgt_v, buf0, buf1, acc, sem0, sem1, semt):
    cid = lax.axis_index("c")
    sid = lax.axis_index("s")
    wid = cid * NS + sid

    pltpu.sync_copy(src_hbm.at[wid], src_v)
    pltpu.sync_copy(tgt_hbm.at[wid, pl.ds(0, GRP)], tgt_v.at[0])

    row0 = sid * RPT
    pltpu.sync_copy(table_hbm.at[pl.ds(ZROW, RPT)], acc.at[pl.ds(row0, RPT)])
    plsc.subcore_barrier()

    bufs = (buf0, buf1)
    sems = (sem0, sem1)
    ngrp = NCH // GRP

    pltpu.async_copy(table_hbm.at[src_v.at[0]], buf0, sem0)

    def group(g, carry):
        @pl.when(g + 1 < ngrp)
        def _():
            pltpu.async_copy(
                tgt_hbm.at[wid, pl.ds((g + 1) * GRP, GRP)],
                tgt_v.at[(g + 1) % 2], semt)

        for p in range(GRP):
            j = g * GRP + p
            @pl.when(j + 1 < NCH)
            def _():
                pltpu.async_copy(
                    table_hbm.at[src_v.at[j + 1]],
                    bufs[(p + 1) % 2], sems[(p + 1) % 2])

            pltpu.make_async_copy(
                table_hbm.at[pl.ds(0, CH)], bufs[p % 2], sems[p % 2]).wait()

            pltpu.sync_copy(bufs[p % 2], acc.at[tgt_v.at[g % 2, p]], add=True)

        @pl.when(g + 1 < ngrp)
        def _():
            pltpu.make_async_copy(
                tgt_hbm.at[wid, pl.ds(0, GRP)], tgt_v.at[(g + 1) % 2],
                semt).wait()
        return carry

    lax.fori_loop(0, ngrp, group, 0)

    plsc.subcore_barrier()
    pltpu.sync_copy(acc.at[pl.ds(row0, RPT)],
                    out_hbm.at[cid, pl.ds(row0, RPT)])


_sc_scatter = functools.partial(
    pl.kernel,
    out_type=jax.ShapeDtypeStruct((NC, N_PAD, DIM), jnp.float32),
    mesh=plsc.VectorSubcoreMesh(core_axis_name="c", subcore_axis_name="s"),
    scratch_types=[
        pltpu.VMEM((NCH, CH), jnp.int32),
        pltpu.VMEM((2, GRP, CH), jnp.int32),
        pltpu.VMEM((CH, DIM), jnp.float32),
        pltpu.VMEM((CH, DIM), jnp.float32),
        pltpu.VMEM_SHARED((N_PAD, DIM), jnp.float32),
        pltpu.SemaphoreType.DMA,
        pltpu.SemaphoreType.DMA,
        pltpu.SemaphoreType.DMA,
    ],
)(_sc_body)


def _merge_body(p_ref, o_ref):
    o_ref[...] = p_ref[0] + p_ref[1]


def _merge(parts):
    return pl.pallas_call(
        _merge_body,
        grid=(NB,),
        in_specs=[pl.BlockSpec((NC, MM_BLK, DIM), lambda i: (0, i, 0))],
        out_specs=pl.BlockSpec((MM_BLK, DIM), lambda i: (i, 0)),
        out_shape=jax.ShapeDtypeStruct((N, DIM), jnp.float32),
    )(parts)


def kernel(edge_lists, node_states, W, b):
    edge_lists = edge_lists.astype(jnp.int32)
    table = _build_table(node_states, W, b.reshape(T, 1, DIM))

    src = edge_lists[:, :, 0] + (jnp.arange(T, dtype=jnp.int32) * NROWS)[:, None]
    tgt = edge_lists[:, :, 1]
    pad = NW * EPW - E
    pad_ids = jnp.arange(pad, dtype=jnp.int32)
    src_w = jnp.concatenate(
        [src.reshape(-1), ZROW + pad_ids % MM_BLK]).reshape(NW, NCH, CH)
    tgt_w = jnp.concatenate(
        [tgt.reshape(-1), pad_ids % N]).reshape(NW, NCH, CH)

    parts = _sc_scatter(table, src_w, tgt_w)
    return _merge(parts)


# speedup vs baseline: 8.6836x; 1.0011x over previous
import functools

import jax
import jax.numpy as jnp
from jax import lax
from jax.experimental import pallas as pl
from jax.experimental.pallas import tpu as pltpu
from jax.experimental.pallas import tpu_sc as plsc

T = 4
DIM = 128
N = 10000
M = 80000

NC = 2
NS = 16
NW = NC * NS
CH = 128
GRP = 16
E = T * M
NCH = 80
EPW = NCH * CH

NB = 10
MM_BLK = N // NB
NROWS = (NB + 1) * MM_BLK
ZROW = N
N_PAD = 10240
RPT = N_PAD // NS


def _mm_body(ns_ref, w_ref, b_ref, out_ref):
    i = pl.program_id(0)

    @pl.when(i < NB)
    def _():
        out_ref[...] = lax.dot_general(
            ns_ref[...], w_ref[...], (((1,), (1,)), ((), ())),
            preferred_element_type=jnp.float32) + b_ref[0]

    @pl.when(i >= NB)
    def _():
        out_ref[...] = jnp.zeros_like(out_ref)


def _build_table(node_states, W, b2d):
    return pl.pallas_call(
        _mm_body,
        grid=(NB + 1, T),
        in_specs=[
            pl.BlockSpec((MM_BLK, DIM), lambda i, t: (jnp.minimum(i, NB - 1), 0)),
            pl.BlockSpec((DIM, DIM), lambda i, t: (t, 0)),
            pl.BlockSpec((1, 1, DIM), lambda i, t: (t, 0, 0)),
        ],
        out_specs=pl.BlockSpec((MM_BLK, DIM), lambda i, t: (t * (NB + 1) + i, 0)),
        out_shape=jax.ShapeDtypeStruct((T * NROWS, DIM), jnp.float32),
    )(node_states, W, b2d)


def _sc_body(table_hbm, src_hbm, tgt_hbm, out_hbm,
             src_v, tgt_v, buf0, buf1, acc, sem0, sem1, semt):
    cid = lax.axis_index("c")
    sid = lax.axis_index("s")
    wid = cid * NS + sid

    pltpu.sync_copy(src_hbm.at[wid], src_v)
    pltpu.sync_copy(tgt_hbm.at[wid, pl.ds(0, GRP)], tgt_v.at[0])

    row0 = sid * RPT
    pltpu.sync_copy(table_hbm.at[pl.ds(ZROW, RPT)], acc.at[pl.ds(row0, RPT)])
    plsc.subcore_barrier()

    bufs = (buf0, buf1)
    sems = (sem0, sem1)
    ngrp = NCH // GRP

    pltpu.async_copy(table_hbm.at[src_v.at[0]], buf0, sem0)

    def group(g, carry):
        @pl.when(g + 1 < ngrp)
        def _():
            pltpu.async_copy(
                tgt_hbm.at[wid, pl.ds((g + 1) * GRP, GRP)],
                tgt_v.at[(g + 1) % 2], semt)

        for p in range(GRP):
            j = g * GRP + p
            @pl.when(j + 1 < NCH)
            def _():
                pltpu.async_copy(
                    table_hbm.at[src_v.at[j + 1, pl.ds(0, CH // 2)]],
                    bufs[(p + 1) % 2].at[pl.ds(0, CH // 2)],
                    sems[(p + 1) % 2])
                pltpu.async_copy(
                    table_hbm.at[src_v.at[j + 1, pl.ds(CH // 2, CH // 2)]],
                    bufs[(p + 1) % 2].at[pl.ds(CH // 2, CH // 2)],
                    sems[(p + 1) % 2])

            pltpu.make_async_copy(
                table_hbm.at[pl.ds(0, CH)], bufs[p % 2], sems[p % 2]).wait()

            pltpu.sync_copy(bufs[p % 2], acc.at[tgt_v.at[g % 2, p]], add=True)

        @pl.when(g + 1 < ngrp)
        def _():
            pltpu.make_async_copy(
                tgt_hbm.at[wid, pl.ds(0, GRP)], tgt_v.at[(g + 1) % 2],
                semt).wait()
        return carry

    lax.fori_loop(0, ngrp, group, 0)

    plsc.subcore_barrier()
    pltpu.sync_copy(acc.at[pl.ds(row0, RPT)],
                    out_hbm.at[cid, pl.ds(row0, RPT)])


_sc_scatter = functools.partial(
    pl.kernel,
    out_type=jax.ShapeDtypeStruct((NC, N_PAD, DIM), jnp.float32),
    mesh=plsc.VectorSubcoreMesh(core_axis_name="c", subcore_axis_name="s"),
    scratch_types=[
        pltpu.VMEM((NCH, CH), jnp.int32),
        pltpu.VMEM((2, GRP, CH), jnp.int32),
        pltpu.VMEM((CH, DIM), jnp.float32),
        pltpu.VMEM((CH, DIM), jnp.float32),
        pltpu.VMEM_SHARED((N_PAD, DIM), jnp.float32),
        pltpu.SemaphoreType.DMA,
        pltpu.SemaphoreType.DMA,
        pltpu.SemaphoreType.DMA,
    ],
)(_sc_body)


def _merge_body(p_ref, o_ref):
    o_ref[...] = p_ref[0] + p_ref[1]


def _merge(parts):
    return pl.pallas_call(
        _merge_body,
        grid=(NB,),
        in_specs=[pl.BlockSpec((NC, MM_BLK, DIM), lambda i: (0, i, 0))],
        out_specs=pl.BlockSpec((MM_BLK, DIM), lambda i: (i, 0)),
        out_shape=jax.ShapeDtypeStruct((N, DIM), jnp.float32),
    )(parts)


def kernel(edge_lists, node_states, W, b):
    edge_lists = edge_lists.astype(jnp.int32)
    table = _build_table(node_states, W, b.reshape(T, 1, DIM))

    src = edge_lists[:, :, 0] + (jnp.arange(T, dtype=jnp.int32) * NROWS)[:, None]
    tgt = edge_lists[:, :, 1]
    pad = NW * EPW - E
    pad_ids = jnp.arange(pad, dtype=jnp.int32)
    src_w = jnp.concatenate(
        [src.reshape(-1), ZROW + pad_ids % MM_BLK]).reshape(NW, NCH, CH)
    tgt_w = jnp.concatenate(
        [tgt.reshape(-1), pad_ids % N]).reshape(NW, NCH, CH)

    parts = _sc_scatter(table, src_w, tgt_w)
    return _merge(parts)


# speedup vs baseline: 9.3483x; 1.0765x over previous
import functools

import jax
import jax.numpy as jnp
from jax import lax
from jax.experimental import pallas as pl
from jax.experimental.pallas import tpu as pltpu
from jax.experimental.pallas import tpu_sc as plsc

T = 4
DIM = 128
N = 10000
M = 80000

NC = 2
NS = 16
NW = NC * NS
CH = 128
GRP = 16
E = T * M
NCH = 80
EPW = NCH * CH

NB = 5
MM_BLK = N // NB
NROWS = (NB + 1) * MM_BLK
ZROW = N
N_PAD = 10240
RPT = N_PAD // NS


def _mm_body(ns_ref, w_ref, b_ref, out_ref):
    i = pl.program_id(0)

    @pl.when(i < NB)
    def _():
        out_ref[...] = lax.dot_general(
            ns_ref[...], w_ref[...], (((1,), (1,)), ((), ())),
            preferred_element_type=jnp.float32) + b_ref[0]

    @pl.when(i >= NB)
    def _():
        out_ref[...] = jnp.zeros_like(out_ref)


def _build_table(node_states, W, b2d):
    return pl.pallas_call(
        _mm_body,
        grid=(NB + 1, T),
        in_specs=[
            pl.BlockSpec((MM_BLK, DIM), lambda i, t: (jnp.minimum(i, NB - 1), 0)),
            pl.BlockSpec((DIM, DIM), lambda i, t: (t, 0)),
            pl.BlockSpec((1, 1, DIM), lambda i, t: (t, 0, 0)),
        ],
        out_specs=pl.BlockSpec((MM_BLK, DIM), lambda i, t: (t * (NB + 1) + i, 0)),
        out_shape=jax.ShapeDtypeStruct((T * NROWS, DIM), jnp.float32),
    )(node_states, W, b2d)


def _sc_body(table_hbm, src_hbm, tgt_hbm, out_hbm,
             src_v, tgt_v, buf0, buf1, acc, sem0, sem1, semt):
    cid = lax.axis_index("c")
    sid = lax.axis_index("s")
    wid = cid * NS + sid

    pltpu.sync_copy(src_hbm.at[wid], src_v)
    pltpu.sync_copy(tgt_hbm.at[wid, pl.ds(0, GRP)], tgt_v.at[0])

    row0 = sid * RPT
    pltpu.sync_copy(table_hbm.at[pl.ds(ZROW, RPT)], acc.at[pl.ds(row0, RPT)])
    plsc.subcore_barrier()

    bufs = (buf0, buf1)
    sems = (sem0, sem1)
    ngrp = NCH // GRP

    pltpu.async_copy(table_hbm.at[src_v.at[0]], buf0, sem0)

    def group(g, carry):
        @pl.when(g + 1 < ngrp)
        def _():
            pltpu.async_copy(
                tgt_hbm.at[wid, pl.ds((g + 1) * GRP, GRP)],
                tgt_v.at[(g + 1) % 2], semt)

        for p in range(GRP):
            j = g * GRP + p
            @pl.when(j + 1 < NCH)
            def _():
                pltpu.async_copy(
                    table_hbm.at[src_v.at[j + 1]],
                    bufs[(p + 1) % 2], sems[(p + 1) % 2])

            pltpu.make_async_copy(
                table_hbm.at[pl.ds(0, CH)], bufs[p % 2], sems[p % 2]).wait()

            pltpu.sync_copy(bufs[p % 2], acc.at[tgt_v.at[g % 2, p]], add=True)

        @pl.when(g + 1 < ngrp)
        def _():
            pltpu.make_async_copy(
                tgt_hbm.at[wid, pl.ds(0, GRP)], tgt_v.at[(g + 1) % 2],
                semt).wait()
        return carry

    lax.fori_loop(0, ngrp, group, 0)

    plsc.subcore_barrier()
    pltpu.sync_copy(acc.at[pl.ds(row0, RPT)],
                    out_hbm.at[cid, pl.ds(row0, RPT)])


_sc_scatter = functools.partial(
    pl.kernel,
    out_type=jax.ShapeDtypeStruct((NC, N_PAD, DIM), jnp.float32),
    mesh=plsc.VectorSubcoreMesh(core_axis_name="c", subcore_axis_name="s"),
    scratch_types=[
        pltpu.VMEM((NCH, CH), jnp.int32),
        pltpu.VMEM((2, GRP, CH), jnp.int32),
        pltpu.VMEM((CH, DIM), jnp.float32),
        pltpu.VMEM((CH, DIM), jnp.float32),
        pltpu.VMEM_SHARED((N_PAD, DIM), jnp.float32),
        pltpu.SemaphoreType.DMA,
        pltpu.SemaphoreType.DMA,
        pltpu.SemaphoreType.DMA,
    ],
)(_sc_body)


def _merge_body(p_ref, o_ref):
    o_ref[...] = p_ref[0] + p_ref[1]


def _merge(parts):
    return pl.pallas_call(
        _merge_body,
        grid=(NB,),
        in_specs=[pl.BlockSpec((NC, MM_BLK, DIM), lambda i: (0, i, 0))],
        out_specs=pl.BlockSpec((MM_BLK, DIM), lambda i: (i, 0)),
        out_shape=jax.ShapeDtypeStruct((N, DIM), jnp.float32),
    )(parts)


def kernel(edge_lists, node_states, W, b):
    edge_lists = edge_lists.astype(jnp.int32)
    table = _build_table(node_states, W, b.reshape(T, 1, DIM))

    src = edge_lists[:, :, 0] + (jnp.arange(T, dtype=jnp.int32) * NROWS)[:, None]
    tgt = edge_lists[:, :, 1]
    pad = NW * EPW - E
    pad_ids = jnp.arange(pad, dtype=jnp.int32)
    src_w = jnp.concatenate(
        [src.reshape(-1), ZROW + pad_ids % MM_BLK]).reshape(NW, NCH, CH)
    tgt_w = jnp.concatenate(
        [tgt.reshape(-1), pad_ids % N]).reshape(NW, NCH, CH)

    parts = _sc_scatter(table, src_w, tgt_w)
    return _merge(parts)


# speedup vs baseline: 9.4025x; 1.0058x over previous
import functools

import jax
import jax.numpy as jnp
from jax import lax
from jax.experimental import pallas as pl
from jax.experimental.pallas import tpu as pltpu
from jax.experimental.pallas import tpu_sc as plsc

T = 4
DIM = 128
N = 10000
M = 80000

NC = 2
NS = 16
NW = NC * NS
CH = 128
GRP = 16
E = T * M
NCH = 80
EPW = NCH * CH

NB = 5
MM_BLK = N // NB
NROWS = N
ZROW = T * N
N_PAD = 10240
RPT = N_PAD // NS


def _mm_body(ns_ref, w_ref, b_ref, out_ref):
    i = pl.program_id(0)

    @pl.when(i < NB)
    def _():
        out_ref[...] = lax.dot_general(
            ns_ref[...], w_ref[...], (((1,), (1,)), ((), ())),
            preferred_element_type=jnp.float32) + b_ref[0]

    @pl.when(i >= NB)
    def _():
        out_ref[...] = jnp.zeros_like(out_ref)


def _build_table(node_states, W, b2d):
    return pl.pallas_call(
        _mm_body,
        grid=(NB + 1, T),
        in_specs=[
            pl.BlockSpec((MM_BLK, DIM), lambda i, t: (jnp.minimum(i, NB - 1), 0)),
            pl.BlockSpec((DIM, DIM), lambda i, t: (t, 0)),
            pl.BlockSpec((1, 1, DIM), lambda i, t: (t, 0, 0)),
        ],
        out_specs=pl.BlockSpec(
            (MM_BLK, DIM),
            lambda i, t: (jnp.where(i < NB, t * NB + i, T * NB), 0)),
        out_shape=jax.ShapeDtypeStruct(((T * NB + 1) * MM_BLK, DIM), jnp.float32),
    )(node_states, W, b2d)


def _sc_body(table_hbm, src_hbm, tgt_hbm, out_hbm,
             src_v, tgt_v, buf0, buf1, acc, sem0, sem1, semt):
    cid = lax.axis_index("c")
    sid = lax.axis_index("s")
    wid = cid * NS + sid

    pltpu.sync_copy(src_hbm.at[wid], src_v)
    pltpu.sync_copy(tgt_hbm.at[wid, pl.ds(0, GRP)], tgt_v.at[0])

    row0 = sid * RPT
    pltpu.sync_copy(table_hbm.at[pl.ds(ZROW, RPT)], acc.at[pl.ds(row0, RPT)])
    plsc.subcore_barrier()

    bufs = (buf0, buf1)
    sems = (sem0, sem1)
    ngrp = NCH // GRP

    pltpu.async_copy(table_hbm.at[src_v.at[0]], buf0, sem0)

    def group(g, carry):
        @pl.when(g + 1 < ngrp)
        def _():
            pltpu.async_copy(
                tgt_hbm.at[wid, pl.ds((g + 1) * GRP, GRP)],
                tgt_v.at[(g + 1) % 2], semt)

        for p in range(GRP):
            j = g * GRP + p
            @pl.when(j + 1 < NCH)
            def _():
                pltpu.async_copy(
                    table_hbm.at[src_v.at[j + 1]],
                    bufs[(p + 1) % 2], sems[(p + 1) % 2])

            pltpu.make_async_copy(
                table_hbm.at[pl.ds(0, CH)], bufs[p % 2], sems[p % 2]).wait()

            pltpu.sync_copy(bufs[p % 2], acc.at[tgt_v.at[g % 2, p]], add=True)

        @pl.when(g + 1 < ngrp)
        def _():
            pltpu.make_async_copy(
                tgt_hbm.at[wid, pl.ds(0, GRP)], tgt_v.at[(g + 1) % 2],
                semt).wait()
        return carry

    lax.fori_loop(0, ngrp, group, 0)

    plsc.subcore_barrier()
    pltpu.sync_copy(acc.at[pl.ds(row0, RPT)],
                    out_hbm.at[cid, pl.ds(row0, RPT)])


_sc_scatter = functools.partial(
    pl.kernel,
    out_type=jax.ShapeDtypeStruct((NC, N_PAD, DIM), jnp.float32),
    mesh=plsc.VectorSubcoreMesh(core_axis_name="c", subcore_axis_name="s"),
    scratch_types=[
        pltpu.VMEM((NCH, CH), jnp.int32),
        pltpu.VMEM((2, GRP, CH), jnp.int32),
        pltpu.VMEM((CH, DIM), jnp.float32),
        pltpu.VMEM((CH, DIM), jnp.float32),
        pltpu.VMEM_SHARED((N_PAD, DIM), jnp.float32),
        pltpu.SemaphoreType.DMA,
        pltpu.SemaphoreType.DMA,
        pltpu.SemaphoreType.DMA,
    ],
)(_sc_body)


def _merge_body(p_ref, o_ref):
    o_ref[...] = p_ref[0] + p_ref[1]


def _merge(parts):
    return pl.pallas_call(
        _merge_body,
        grid=(NB,),
        in_specs=[pl.BlockSpec((NC, MM_BLK, DIM), lambda i: (0, i, 0))],
        out_specs=pl.BlockSpec((MM_BLK, DIM), lambda i: (i, 0)),
        out_shape=jax.ShapeDtypeStruct((N, DIM), jnp.float32),
    )(parts)


def kernel(edge_lists, node_states, W, b):
    edge_lists = edge_lists.astype(jnp.int32)
    table = _build_table(node_states, W, b.reshape(T, 1, DIM))

    src = edge_lists[:, :, 0] + (jnp.arange(T, dtype=jnp.int32) * NROWS)[:, None]
    tgt = edge_lists[:, :, 1]
    pad = NW * EPW - E
    pad_ids = jnp.arange(pad, dtype=jnp.int32)
    src_w = jnp.concatenate(
        [src.reshape(-1), ZROW + pad_ids % MM_BLK]).reshape(NW, NCH, CH)
    tgt_w = jnp.concatenate(
        [tgt.reshape(-1), pad_ids % N]).reshape(NW, NCH, CH)

    parts = _sc_scatter(table, src_w, tgt_w)
    return _merge(parts)


# speedup vs baseline: 9.7577x; 1.0378x over previous
import functools

import jax
import jax.numpy as jnp
from jax import lax
from jax.experimental import pallas as pl
from jax.experimental.pallas import tpu as pltpu
from jax.experimental.pallas import tpu_sc as plsc

T = 4
DIM = 128
N = 10000
M = 80000

NC = 2
NS = 16
NW = NC * NS
CH = 128
GRP = 16
E = T * M
NCH = 80
EPW = NCH * CH

NB = 2
MM_BLK = N // NB
NROWS = N
ZROW = T * N
N_PAD = 10240
RPT = N_PAD // NS


def _mm_body(ns_ref, w_ref, b_ref, out_ref):
    i = pl.program_id(0)

    @pl.when(i < NB)
    def _():
        out_ref[...] = lax.dot_general(
            ns_ref[...], w_ref[...], (((1,), (1,)), ((), ())),
            preferred_element_type=jnp.float32) + b_ref[0]

    @pl.when(i >= NB)
    def _():
        out_ref[...] = jnp.zeros_like(out_ref)


def _build_table(node_states, W, b2d):
    return pl.pallas_call(
        _mm_body,
        grid=(NB + 1, T),
        in_specs=[
            pl.BlockSpec((MM_BLK, DIM), lambda i, t: (jnp.minimum(i, NB - 1), 0)),
            pl.BlockSpec((DIM, DIM), lambda i, t: (t, 0)),
            pl.BlockSpec((1, 1, DIM), lambda i, t: (t, 0, 0)),
        ],
        out_specs=pl.BlockSpec(
            (MM_BLK, DIM),
            lambda i, t: (jnp.where(i < NB, t * NB + i, T * NB), 0)),
        out_shape=jax.ShapeDtypeStruct(((T * NB + 1) * MM_BLK, DIM), jnp.float32),
    )(node_states, W, b2d)


def _sc_body(table_hbm, src_hbm, tgt_hbm, out_hbm,
             src_v, tgt_v, buf0, buf1, acc, sem0, sem1, semt):
    cid = lax.axis_index("c")
    sid = lax.axis_index("s")
    wid = cid * NS + sid

    pltpu.sync_copy(src_hbm.at[wid], src_v)
    pltpu.sync_copy(tgt_hbm.at[wid, pl.ds(0, GRP)], tgt_v.at[0])

    row0 = sid * RPT
    pltpu.sync_copy(table_hbm.at[pl.ds(ZROW, RPT)], acc.at[pl.ds(row0, RPT)])
    plsc.subcore_barrier()

    bufs = (buf0, buf1)
    sems = (sem0, sem1)
    ngrp = NCH // GRP

    pltpu.async_copy(table_hbm.at[src_v.at[0]], buf0, sem0)

    def group(g, carry):
        @pl.when(g + 1 < ngrp)
        def _():
            pltpu.async_copy(
                tgt_hbm.at[wid, pl.ds((g + 1) * GRP, GRP)],
                tgt_v.at[(g + 1) % 2], semt)

        for p in range(GRP):
            j = g * GRP + p
            @pl.when(j + 1 < NCH)
            def _():
                pltpu.async_copy(
                    table_hbm.at[src_v.at[j + 1]],
                    bufs[(p + 1) % 2], sems[(p + 1) % 2])

            pltpu.make_async_copy(
                table_hbm.at[pl.ds(0, CH)], bufs[p % 2], sems[p % 2]).wait()

            pltpu.sync_copy(bufs[p % 2], acc.at[tgt_v.at[g % 2, p]], add=True)

        @pl.when(g + 1 < ngrp)
        def _():
            pltpu.make_async_copy(
                tgt_hbm.at[wid, pl.ds(0, GRP)], tgt_v.at[(g + 1) % 2],
                semt).wait()
        return carry

    lax.fori_loop(0, ngrp, group, 0)

    plsc.subcore_barrier()
    pltpu.sync_copy(acc.at[pl.ds(row0, RPT)],
                    out_hbm.at[cid, pl.ds(row0, RPT)])


_sc_scatter = functools.partial(
    pl.kernel,
    out_type=jax.ShapeDtypeStruct((NC, N_PAD, DIM), jnp.float32),
    mesh=plsc.VectorSubcoreMesh(core_axis_name="c", subcore_axis_name="s"),
    scratch_types=[
        pltpu.VMEM((NCH, CH), jnp.int32),
        pltpu.VMEM((2, GRP, CH), jnp.int32),
        pltpu.VMEM((CH, DIM), jnp.float32),
        pltpu.VMEM((CH, DIM), jnp.float32),
        pltpu.VMEM_SHARED((N_PAD, DIM), jnp.float32),
        pltpu.SemaphoreType.DMA,
        pltpu.SemaphoreType.DMA,
        pltpu.SemaphoreType.DMA,
    ],
)(_sc_body)


def _merge_body(p_ref, o_ref):
    o_ref[...] = p_ref[0] + p_ref[1]


def _merge(parts):
    return pl.pallas_call(
        _merge_body,
        grid=(NB,),
        in_specs=[pl.BlockSpec((NC, MM_BLK, DIM), lambda i: (0, i, 0))],
        out_specs=pl.BlockSpec((MM_BLK, DIM), lambda i: (i, 0)),
        out_shape=jax.ShapeDtypeStruct((N, DIM), jnp.float32),
    )(parts)


def kernel(edge_lists, node_states, W, b):
    edge_lists = edge_lists.astype(jnp.int32)
    table = _build_table(node_states, W, b.reshape(T, 1, DIM))

    src = edge_lists[:, :, 0] + (jnp.arange(T, dtype=jnp.int32) * NROWS)[:, None]
    tgt = edge_lists[:, :, 1]
    pad = NW * EPW - E
    pad_ids = jnp.arange(pad, dtype=jnp.int32)
    src_w = jnp.concatenate(
        [src.reshape(-1), ZROW + pad_ids % MM_BLK]).reshape(NW, NCH, CH)
    tgt_w = jnp.concatenate(
        [tgt.reshape(-1), pad_ids % N]).reshape(NW, NCH, CH)

    parts = _sc_scatter(table, src_w, tgt_w)
    return _merge(parts)
